# baseline (speedup 1.0000x reference)
import jax
import jax.numpy as jnp
from jax.experimental import pallas as pl

N_CLASSES = 1000
BATCH = 16384
B_BLK = 256


def _onehot_body(idx_ref, out_ref):
    idx = idx_ref[0, 0, :].astype(jnp.int32)
    cls = jax.lax.broadcasted_iota(jnp.int32, (B_BLK, N_CLASSES), 1)
    out_ref[...] = (idx[:, None] == cls).astype(jnp.float32)


def kernel(inputs):
    idx3 = inputs.astype(jnp.int32).reshape(BATCH // B_BLK, 1, B_BLK)
    return pl.pallas_call(
        _onehot_body,
        grid=(BATCH // B_BLK,),
        in_specs=[pl.BlockSpec((1, 1, B_BLK), lambda i: (i, 0, 0))],
        out_specs=pl.BlockSpec((B_BLK, N_CLASSES), lambda i: (i, 0)),
        out_shape=jax.ShapeDtypeStruct((BATCH, N_CLASSES), jnp.float32),
    )(idx3)


# speedup vs baseline: 1.2174x; 1.2174x over previous
import jax
import jax.numpy as jnp
from jax.experimental import pallas as pl

N_CLASSES = 1000
BATCH = 16384
B_BLK = 2048


def _onehot_body(idx_ref, out_ref):
    idx = idx_ref[0, 0, :].astype(jnp.int32)
    cls = jax.lax.broadcasted_iota(jnp.int32, (B_BLK, N_CLASSES), 1)
    out_ref[...] = (idx[:, None] == cls).astype(jnp.float32)


def kernel(inputs):
    idx3 = inputs.astype(jnp.int32).reshape(BATCH // B_BLK, 1, B_BLK)
    return pl.pallas_call(
        _onehot_body,
        grid=(BATCH // B_BLK,),
        in_specs=[pl.BlockSpec((1, 1, B_BLK), lambda i: (i, 0, 0))],
        out_specs=pl.BlockSpec((B_BLK, N_CLASSES), lambda i: (i, 0)),
        out_shape=jax.ShapeDtypeStruct((BATCH, N_CLASSES), jnp.float32),
    )(idx3)


# speedup vs baseline: 1.2185x; 1.0009x over previous
import jax
import jax.numpy as jnp
from jax.experimental import pallas as pl
from jax.experimental.pallas import tpu as pltpu

N_CLASSES = 1000
BATCH = 16384
B_BLK = 512
NBUF = 8
GRID = BATCH // B_BLK


def _onehot_body(idx_ref, out_hbm, scratch, sems):
    i = pl.program_id(0)
    slot = jax.lax.rem(i, NBUF)

    @pl.when(i >= NBUF)
    def _wait_prev():
        j = i - NBUF
        pltpu.make_async_copy(
            scratch.at[jax.lax.rem(j, NBUF)],
            out_hbm.at[pl.ds(j * B_BLK, B_BLK), :],
            sems.at[jax.lax.rem(j, NBUF)],
        ).wait()

    idx = idx_ref[0, 0, :].astype(jnp.int32)
    cls = jax.lax.broadcasted_iota(jnp.int32, (B_BLK, N_CLASSES), 1)
    scratch[slot] = (idx[:, None] == cls).astype(jnp.float32)

    pltpu.make_async_copy(
        scratch.at[slot],
        out_hbm.at[pl.ds(i * B_BLK, B_BLK), :],
        sems.at[slot],
    ).start()

    @pl.when(i == GRID - 1)
    def _drain_tail():
        for k in range(NBUF):
            j = GRID - NBUF + k
            pltpu.make_async_copy(
                scratch.at[jax.lax.rem(jnp.int32(j), NBUF)],
                out_hbm.at[pl.ds(j * B_BLK, B_BLK), :],
                sems.at[jax.lax.rem(jnp.int32(j), NBUF)],
            ).wait()


def kernel(inputs):
    idx3 = inputs.astype(jnp.int32).reshape(GRID, 1, B_BLK)
    return pl.pallas_call(
        _onehot_body,
        grid=(GRID,),
        in_specs=[pl.BlockSpec((1, 1, B_BLK), lambda i: (i, 0, 0))],
        out_specs=pl.BlockSpec(memory_space=pl.ANY),
        out_shape=jax.ShapeDtypeStruct((BATCH, N_CLASSES), jnp.float32),
        scratch_shapes=[
            pltpu.VMEM((NBUF, B_BLK, N_CLASSES), jnp.float32),
            pltpu.SemaphoreType.DMA((NBUF,)),
        ],
    )(idx3)


# speedup vs baseline: 4.4564x; 3.6572x over previous
import jax
import jax.numpy as jnp
from jax.experimental import pallas as pl

N_CLASSES = 1000
BATCH = 16384
B_BLK = 2048
GRID = BATCH // B_BLK


def _onehot_body(idx_ref, out_ref):
    idx = idx_ref[0, 0, :].astype(jnp.int32)
    cls = jax.lax.broadcasted_iota(jnp.int32, (N_CLASSES, B_BLK), 0)
    out_ref[...] = (cls == idx[None, :]).astype(jnp.float32)


def kernel(inputs):
    idx3 = inputs.astype(jnp.int32).reshape(GRID, 1, B_BLK)
    out_t = pl.pallas_call(
        _onehot_body,
        grid=(GRID,),
        in_specs=[pl.BlockSpec((1, 1, B_BLK), lambda i: (i, 0, 0))],
        out_specs=pl.BlockSpec((N_CLASSES, B_BLK), lambda i: (0, i)),
        out_shape=jax.ShapeDtypeStruct((N_CLASSES, BATCH), jnp.float32),
    )(idx3)
    return out_t.T
